# baseline (speedup 1.0000x reference)
import functools

import jax
import jax.numpy as jnp
from jax import lax
from jax.experimental import pallas as pl
from jax.experimental.pallas import tpu as pltpu
from jax.experimental.pallas import tpu_sc as plsc

_NC = 2
_NS = 16
_LANES = 16


def _pick_chunk(per_worker):
    for b in range(40, 7, -8):
        if per_worker % b == 0:
            return b
    return None


def _edge_stage(k_arr, qv_arr, src1, dst1, zeros_nd):
    n, d = k_arr.shape
    e = src1.shape[0]
    nw = _NC * _NS
    per_w = e // nw
    b = _pick_chunk(per_w)
    chunks = per_w // b
    rows_main = (n // _NS) & ~7
    rows_rem = n - _NS * rows_main
    nchunk = d // _LANES

    mesh = plsc.VectorSubcoreMesh(
        core_axis_name="c", subcore_axis_name="s",
        num_cores=_NC, num_subcores=_NS)

    @functools.partial(
        pl.kernel,
        mesh=mesh,
        out_type=(
            jax.ShapeDtypeStruct((n, d), jnp.float32),
            jax.ShapeDtypeStruct((n, d), jnp.float32),
        ),
        scratch_types=[
            pltpu.VMEM((2 * b,), jnp.int32),
            pltpu.VMEM((2 * b,), jnp.int32),
            pltpu.VMEM((2 * b,), jnp.int32),
            pltpu.VMEM((2 * b,), jnp.int32),
            pltpu.VMEM((2 * b,), jnp.int32),
            pltpu.VMEM((b, 2 * d), jnp.float32),
            pltpu.VMEM((b, 2 * d), jnp.float32),
            pltpu.VMEM((b, d), jnp.float32),
            pltpu.VMEM((b, d), jnp.float32),
            pltpu.VMEM((b, d), jnp.float32),
            pltpu.VMEM((b, d), jnp.float32),
            pltpu.VMEM_SHARED((n, d), jnp.float32),
            pltpu.SemaphoreType.DMA,
            pltpu.SemaphoreType.DMA,
            pltpu.SemaphoreType.DMA,
            pltpu.SemaphoreType.DMA,
            pltpu.SemaphoreType.DMA,
            pltpu.SemaphoreType.DMA,
            pltpu.SemaphoreType.DMA,
            pltpu.SemaphoreType.DMA,
            pltpu.SemaphoreType.DMA,
        ],
    )
    def edge_kernel(k_hbm, qv_hbm, sd_hbm, zero_hbm,
                    out0, out1,
                    sdb0, sdb1, sdb2, sdb3, sdb4,
                    qvb0, qvb1, kb0, kb1, ob0, ob1, agg_sh,
                    is0, is1, is2, is3, is4,
                    gsem0, gsem1, ssem0, ssem1):
        c = lax.axis_index("c")
        s = lax.axis_index("s")
        wid = s * _NC + c
        r0 = s * rows_main
        sdb = (sdb0, sdb1, sdb2, sdb3, sdb4)
        isem = (is0, is1, is2, is3, is4)
        qvb = (qvb0, qvb1)
        kb = (kb0, kb1)
        ob = (ob0, ob1)
        gsem = (gsem0, gsem1)
        ssem = (ssem0, ssem1)
        pltpu.sync_copy(zero_hbm.at[pl.ds(r0, rows_main)],
                        agg_sh.at[pl.ds(r0, rows_main)])
        if rows_rem:
            @pl.when(s == _NS - 1)
            def _():
                rr = _NS * rows_main
                pltpu.sync_copy(zero_hbm.at[pl.ds(rr, rows_rem)],
                                agg_sh.at[pl.ds(rr, rows_rem)])
        plsc.subcore_barrier()

        base0 = wid * (chunks * 2 * b)

        def fetch_idx(j, isl):
            pltpu.async_copy(sd_hbm.at[pl.ds(base0 + j * 2 * b, 2 * b)],
                             sdb[isl], isem[isl])

        def wait_idx(j, isl):
            pltpu.make_async_copy(sd_hbm.at[pl.ds(base0 + j * 2 * b, 2 * b)],
                                  sdb[isl], isem[isl]).wait()

        def issue_gathers(isl, dsl):
            pltpu.async_copy(qv_hbm.at[sdb[isl].at[pl.ds(b, b)]], qvb[dsl],
                             gsem[dsl])
            pltpu.async_copy(k_hbm.at[sdb[isl].at[pl.ds(0, b)]], kb[dsl],
                             gsem[dsl])

        def wait_gathers(isl, dsl):
            pltpu.make_async_copy(qv_hbm.at[sdb[isl].at[pl.ds(b, b)]],
                                  qvb[dsl], gsem[dsl]).wait()
            pltpu.make_async_copy(k_hbm.at[sdb[isl].at[pl.ds(0, b)]],
                                  kb[dsl], gsem[dsl]).wait()

        def drain_scatter(dsl, isl):
            pltpu.make_async_copy(ob[dsl], agg_sh.at[sdb[isl].at[pl.ds(0, b)]],
                                  ssem[dsl]).wait()

        def turn(j, t):
            dsl = t % 2
            @pl.when(j + 1 < chunks)
            def _():
                wait_idx(j + 1, (t + 1) % 5)
                issue_gathers((t + 1) % 5, 1 - dsl)

            @pl.when(j >= 2)
            def _():
                drain_scatter(dsl, (t - 2) % 5)

            @pl.when(j + 3 < chunks)
            def _():
                fetch_idx(j + 3, (t + 3) % 5)

            wait_gathers(t % 5, dsl)

            @plsc.parallel_loop(0, b, 1, unroll=2)
            def row_body(r):
                for cc in range(nchunk):
                    slc = pl.ds(cc * _LANES, _LANES)
                    tt = kb[dsl][r, slc] + qvb[dsl][r, slc]
                    v = qvb[dsl][r, pl.ds(d + cc * _LANES, _LANES)]
                    ob[dsl][r, slc] = v / (1.0 + jnp.exp(-tt))

            pltpu.async_copy(ob[dsl], agg_sh.at[sdb[t % 5].at[pl.ds(0, b)]],
                             ssem[dsl], add=True)

        fetch_idx(jnp.int32(0), 0)
        fetch_idx(jnp.int32(1), 1)
        fetch_idx(jnp.int32(2), 2)
        wait_idx(jnp.int32(0), 0)
        issue_gathers(0, 0)

        assert chunks % 10 == 0, chunks

        def dec_body(p, carry):
            for t in range(10):
                turn(10 * p + t, t)
            return carry

        lax.fori_loop(0, chunks // 10, dec_body, 0, unroll=False)
        drain_scatter((chunks - 1) % 2, (chunks - 1) % 5)
        drain_scatter((chunks - 2) % 2, (chunks - 2) % 5)
        plsc.subcore_barrier()

        @pl.when(c == 0)
        def _():
            pltpu.sync_copy(agg_sh.at[pl.ds(r0, rows_main)],
                            out0.at[pl.ds(r0, rows_main)])
            if rows_rem:
                @pl.when(s == _NS - 1)
                def _():
                    rr = _NS * rows_main
                    pltpu.sync_copy(agg_sh.at[pl.ds(rr, rows_rem)],
                                    out0.at[pl.ds(rr, rows_rem)])

        @pl.when(c == 1)
        def _():
            pltpu.sync_copy(agg_sh.at[pl.ds(r0, rows_main)],
                            out1.at[pl.ds(r0, rows_main)])
            if rows_rem:
                @pl.when(s == _NS - 1)
                def _():
                    rr = _NS * rows_main
                    pltpu.sync_copy(agg_sh.at[pl.ds(rr, rows_rem)],
                                    out1.at[pl.ds(rr, rows_rem)])

    sd = jnp.stack([dst1.reshape(-1, b), src1.reshape(-1, b)],
                   axis=1).reshape(-1)
    return edge_kernel(k_arr, qv_arr, sd, zeros_nd)


def _proj0_body(d, h_ref, w_ref, b_ref, k_ref, qv_ref, s_ref):
    acc = jnp.dot(h_ref[...], w_ref[...],
                  preferred_element_type=jnp.float32) + b_ref[...]
    k_ref[...] = acc[:, :d]
    qv_ref[...] = acc[:, d:3 * d]
    s_ref[...] = acc[:, 3 * d:]


def _proj0(x, w_all, b_all):
    n, d = x.shape
    gsteps = 5
    br = n // gsteps
    return pl.pallas_call(
        functools.partial(_proj0_body, d),
        grid=(gsteps,),
        in_specs=[
            pl.BlockSpec((br, d), lambda i: (i, 0)),
            pl.BlockSpec((d, 4 * d), lambda i: (0, 0)),
            pl.BlockSpec((1, 4 * d), lambda i: (0, 0)),
        ],
        out_specs=[
            pl.BlockSpec((br, d), lambda i: (i, 0)),
            pl.BlockSpec((br, 2 * d), lambda i: (i, 0)),
            pl.BlockSpec((br, d), lambda i: (i, 0)),
        ],
        out_shape=[
            jax.ShapeDtypeStruct((n, d), jnp.float32),
            jax.ShapeDtypeStruct((n, 2 * d), jnp.float32),
            jax.ShapeDtypeStruct((n, d), jnp.float32),
        ],
    )(x, w_all, b_all)


def _proj1_body(d, a0_ref, a1_ref, sp_ref, bp_ref, w_ref, b_ref,
                k_ref, qv_ref, s_ref):
    h = a0_ref[...] + a1_ref[...] + sp_ref[...] + bp_ref[...]
    h = jnp.maximum(h, 0.0)
    acc = jnp.dot(h, w_ref[...], preferred_element_type=jnp.float32) + b_ref[...]
    k_ref[...] = acc[:, :d]
    qv_ref[...] = acc[:, d:3 * d]
    s_ref[...] = acc[:, 3 * d:]


def _proj1(a0, a1, s_prev, b_prev, w_all, b_all):
    n, d = a0.shape
    gsteps = 5
    br = n // gsteps
    return pl.pallas_call(
        functools.partial(_proj1_body, d),
        grid=(gsteps,),
        in_specs=[
            pl.BlockSpec((br, d), lambda i: (i, 0)),
            pl.BlockSpec((br, d), lambda i: (i, 0)),
            pl.BlockSpec((br, d), lambda i: (i, 0)),
            pl.BlockSpec((1, d), lambda i: (0, 0)),
            pl.BlockSpec((d, 4 * d), lambda i: (0, 0)),
            pl.BlockSpec((1, 4 * d), lambda i: (0, 0)),
        ],
        out_specs=[
            pl.BlockSpec((br, d), lambda i: (i, 0)),
            pl.BlockSpec((br, 2 * d), lambda i: (i, 0)),
            pl.BlockSpec((br, d), lambda i: (i, 0)),
        ],
        out_shape=[
            jax.ShapeDtypeStruct((n, d), jnp.float32),
            jax.ShapeDtypeStruct((n, 2 * d), jnp.float32),
            jax.ShapeDtypeStruct((n, d), jnp.float32),
        ],
    )(a0, a1, s_prev, b_prev, w_all, b_all)


def _pool_head_body(g, a0_ref, a1_ref, sp_ref, bp_ref, batch_ref,
                    w1_ref, b1_ref, w2_ref, b2_ref, w3_ref, b3_ref,
                    out_ref, gmax_ref):
    h = a0_ref[...] + a1_ref[...] + sp_ref[...] + bp_ref[...]
    bt = batch_ref[...]
    seg = lax.broadcasted_iota(jnp.int32, (1, g), 1)
    oh = (bt == seg).astype(jnp.float32)
    gsum = lax.dot_general(oh, h, (((0,), (0,)), ((), ())),
                           preferred_element_type=jnp.float32)
    ones_col = jnp.ones((h.shape[0], 1), jnp.float32)
    cnt = lax.dot_general(oh, ones_col, (((0,), (0,)), ((), ())),
                          preferred_element_type=jnp.float32)
    gmean = gsum / jnp.maximum(cnt, 1.0)

    def seg_max(gi, carry):
        mask = bt == gi
        mg = jnp.max(jnp.where(mask, h, -jnp.inf), axis=0, keepdims=True)
        gmax_ref[pl.ds(gi, 1), :] = mg
        return carry

    lax.fori_loop(0, g, seg_max, 0, unroll=False)
    gmax = gmax_ref[...]
    f = jnp.concatenate([gmax, gmean], axis=1)
    o = jnp.maximum(jnp.dot(f, w1_ref[...],
                            preferred_element_type=jnp.float32) + b1_ref[...],
                    0.0)
    o = jnp.maximum(jnp.dot(o, w2_ref[...],
                            preferred_element_type=jnp.float32) + b2_ref[...],
                    0.0)
    out_ref[...] = jnp.dot(o, w3_ref[...],
                           preferred_element_type=jnp.float32) + b3_ref[...]


def _pool_head(a0, a1, s_prev, b_prev, batch2d, g,
               w1, b1, w2, b2, w3p, b3p):
    n, d = a0.shape
    return pl.pallas_call(
        functools.partial(_pool_head_body, g),
        out_shape=jax.ShapeDtypeStruct((g, d), jnp.float32),
        scratch_shapes=[pltpu.VMEM((g, d), jnp.float32)],
    )(a0, a1, s_prev, b_prev, batch2d, w1, b1, w2, b2, w3p, b3p)


def kernel(x, edge_index, batch, params):
    n, d = x.shape
    src1 = edge_index[0].astype(jnp.int32)
    dst1 = edge_index[1].astype(jnp.int32)
    zeros_nd = jnp.zeros((n, d), jnp.float32)

    num_layers = len([k for k in params if k.startswith('conv')])
    h_k = None
    h_qv = None
    h_s = None
    a0 = a1 = None
    b_prev = None
    for li in range(num_layers):
        p = params['conv%d' % li]
        w_all = jnp.concatenate([p['Wk'], p['Wq'], p['Wv'], p['Ws']], axis=1)
        b_all = jnp.concatenate(
            [p['bk'], p['bq'], p['bv'],
             jnp.zeros_like(p['bk'])]).reshape(1, -1)
        if li == 0:
            h_k, h_qv, h_s = _proj0(x, w_all, b_all)
        else:
            h_k, h_qv, h_s = _proj1(a0, a1, h_s, b_prev, w_all, b_all)
        a0, a1 = _edge_stage(h_k, h_qv, src1, dst1, zeros_nd)
        b_prev = p['b'].reshape(1, -1)

    hp = params['head']
    seg_count = 64
    bn_scale = 1.0 / jnp.sqrt(1.0 + 1e-5)
    w1 = hp['W1'] * (hp['g1'] * bn_scale)[None, :]
    b1 = (hp['b1'] * hp['g1'] * bn_scale + hp['be1']).reshape(1, -1)
    w2 = hp['W2'] * (hp['g2'] * bn_scale)[None, :]
    b2 = (hp['b2'] * hp['g2'] * bn_scale + hp['be2']).reshape(1, -1)
    classes = hp['W3'].shape[1]
    w3p = jnp.zeros((hp['W3'].shape[0], d), jnp.float32)
    w3p = w3p.at[:, :classes].set(hp['W3'])
    b3p = jnp.zeros((1, d), jnp.float32)
    b3p = b3p.at[:, :classes].set(hp['b3'][None, :])

    batch2d = batch.astype(jnp.int32).reshape(-1, 1)
    out = _pool_head(a0, a1, h_s, b_prev, batch2d, seg_count,
                     w1, b1, w2, b2, w3p, b3p)
    return out[:, :classes]


# speedup vs baseline: 1.3519x; 1.3519x over previous
import functools

import jax
import jax.numpy as jnp
from jax import lax
from jax.experimental import pallas as pl
from jax.experimental.pallas import tpu as pltpu
from jax.experimental.pallas import tpu_sc as plsc

_NC = 2
_NS = 16
_LANES = 16


def _pick_chunk(per_worker):
    for b in range(40, 7, -8):
        if per_worker % b == 0:
            return b
    return None


def _edge_stage(k_arr, qv_arr, src1, dst1, zeros_nd):
    n, d = k_arr.shape
    e = src1.shape[0]
    nw = _NC * _NS
    per_w = e // nw
    b = _pick_chunk(per_w)
    chunks = per_w // b
    rows_main = (n // _NS) & ~7
    rows_rem = n - _NS * rows_main
    nchunk = d // _LANES

    mesh = plsc.VectorSubcoreMesh(
        core_axis_name="c", subcore_axis_name="s",
        num_cores=_NC, num_subcores=_NS)

    @functools.partial(
        pl.kernel,
        mesh=mesh,
        out_type=(
            jax.ShapeDtypeStruct((n, d), jnp.float32),
            jax.ShapeDtypeStruct((n, d), jnp.float32),
        ),
        scratch_types=[
            pltpu.VMEM((b,), jnp.int32),
            pltpu.VMEM((b,), jnp.int32),
            pltpu.VMEM((b,), jnp.int32),
            pltpu.VMEM((b,), jnp.int32),
            pltpu.VMEM((b,), jnp.int32),
            pltpu.VMEM((b,), jnp.int32),
            pltpu.VMEM((b,), jnp.int32),
            pltpu.VMEM((b,), jnp.int32),
            pltpu.VMEM((b,), jnp.int32),
            pltpu.VMEM((b,), jnp.int32),
            pltpu.VMEM((b, 2 * d), jnp.float32),
            pltpu.VMEM((b, 2 * d), jnp.float32),
            pltpu.VMEM((b, d), jnp.float32),
            pltpu.VMEM((b, d), jnp.float32),
            pltpu.VMEM((b, d), jnp.float32),
            pltpu.VMEM((b, d), jnp.float32),
            pltpu.VMEM_SHARED((n, d), jnp.float32),
            pltpu.SemaphoreType.DMA,
            pltpu.SemaphoreType.DMA,
            pltpu.SemaphoreType.DMA,
            pltpu.SemaphoreType.DMA,
            pltpu.SemaphoreType.DMA,
            pltpu.SemaphoreType.DMA,
            pltpu.SemaphoreType.DMA,
            pltpu.SemaphoreType.DMA,
            pltpu.SemaphoreType.DMA,
        ],
    )
    def edge_kernel(k_hbm, qv_hbm, src_hbm, dst_hbm, zero_hbm,
                    out0, out1,
                    sb0, sb1, sb2, sb3, sb4,
                    db0, db1, db2, db3, db4,
                    qvb0, qvb1, kb0, kb1, ob0, ob1, agg_sh,
                    is0, is1, is2, is3, is4,
                    gsem0, gsem1, ssem0, ssem1):
        c = lax.axis_index("c")
        s = lax.axis_index("s")
        wid = s * _NC + c
        r0 = s * rows_main
        sb = (sb0, sb1, sb2, sb3, sb4)
        db = (db0, db1, db2, db3, db4)
        isem = (is0, is1, is2, is3, is4)
        qvb = (qvb0, qvb1)
        kb = (kb0, kb1)
        ob = (ob0, ob1)
        gsem = (gsem0, gsem1)
        ssem = (ssem0, ssem1)
        pltpu.sync_copy(zero_hbm.at[pl.ds(r0, rows_main)],
                        agg_sh.at[pl.ds(r0, rows_main)])
        if rows_rem:
            @pl.when(s == _NS - 1)
            def _():
                rr = _NS * rows_main
                pltpu.sync_copy(zero_hbm.at[pl.ds(rr, rows_rem)],
                                agg_sh.at[pl.ds(rr, rows_rem)])
        plsc.subcore_barrier()

        base0 = wid * (chunks * b)

        def fetch_idx(j, isl):
            pltpu.async_copy(src_hbm.at[pl.ds(base0 + j * b, b)], sb[isl],
                             isem[isl])
            pltpu.async_copy(dst_hbm.at[pl.ds(base0 + j * b, b)], db[isl],
                             isem[isl])

        def wait_idx(j, isl):
            pltpu.make_async_copy(src_hbm.at[pl.ds(base0 + j * b, b)],
                                  sb[isl], isem[isl]).wait()
            pltpu.make_async_copy(dst_hbm.at[pl.ds(base0 + j * b, b)],
                                  db[isl], isem[isl]).wait()

        def issue_gathers(isl, dsl):
            pltpu.async_copy(qv_hbm.at[sb[isl]], qvb[dsl], gsem[dsl])
            pltpu.async_copy(k_hbm.at[db[isl]], kb[dsl], gsem[dsl])

        def wait_gathers(isl, dsl):
            pltpu.make_async_copy(qv_hbm.at[sb[isl]], qvb[dsl],
                                  gsem[dsl]).wait()
            pltpu.make_async_copy(k_hbm.at[db[isl]], kb[dsl],
                                  gsem[dsl]).wait()

        def drain_scatter(dsl, isl):
            pltpu.make_async_copy(ob[dsl], agg_sh.at[db[isl]],
                                  ssem[dsl]).wait()

        def turn(j, t):
            dsl = t % 2
            @pl.when(j + 1 < chunks)
            def _():
                wait_idx(j + 1, (t + 1) % 5)
                issue_gathers((t + 1) % 5, 1 - dsl)

            @pl.when(j >= 2)
            def _():
                drain_scatter(dsl, (t - 2) % 5)

            @pl.when(j + 3 < chunks)
            def _():
                fetch_idx(j + 3, (t + 3) % 5)

            wait_gathers(t % 5, dsl)

            assert nchunk & (nchunk - 1) == 0
            csh = nchunk.bit_length() - 1

            @plsc.parallel_loop(0, b * nchunk, 1, unroll=2)
            def row_body(i):
                r = i >> csh
                c16 = (i & (nchunk - 1)) * _LANES
                slc = pl.ds(c16, _LANES)
                tt = kb[dsl][r, slc] + qvb[dsl][r, slc]
                v = qvb[dsl][r, pl.ds(d + c16, _LANES)]
                ob[dsl][r, slc] = v / (1.0 + jnp.exp(-tt))

            pltpu.async_copy(ob[dsl], agg_sh.at[db[t % 5]], ssem[dsl],
                             add=True)

        fetch_idx(jnp.int32(0), 0)
        fetch_idx(jnp.int32(1), 1)
        fetch_idx(jnp.int32(2), 2)
        wait_idx(jnp.int32(0), 0)
        issue_gathers(0, 0)

        assert chunks % 10 == 0, chunks

        def dec_body(p, carry):
            for t in range(10):
                turn(10 * p + t, t)
            return carry

        lax.fori_loop(0, chunks // 10, dec_body, 0, unroll=False)
        drain_scatter((chunks - 1) % 2, (chunks - 1) % 5)
        drain_scatter((chunks - 2) % 2, (chunks - 2) % 5)
        plsc.subcore_barrier()

        @pl.when(c == 0)
        def _():
            pltpu.sync_copy(agg_sh.at[pl.ds(r0, rows_main)],
                            out0.at[pl.ds(r0, rows_main)])
            if rows_rem:
                @pl.when(s == _NS - 1)
                def _():
                    rr = _NS * rows_main
                    pltpu.sync_copy(agg_sh.at[pl.ds(rr, rows_rem)],
                                    out0.at[pl.ds(rr, rows_rem)])

        @pl.when(c == 1)
        def _():
            pltpu.sync_copy(agg_sh.at[pl.ds(r0, rows_main)],
                            out1.at[pl.ds(r0, rows_main)])
            if rows_rem:
                @pl.when(s == _NS - 1)
                def _():
                    rr = _NS * rows_main
                    pltpu.sync_copy(agg_sh.at[pl.ds(rr, rows_rem)],
                                    out1.at[pl.ds(rr, rows_rem)])

    return edge_kernel(k_arr, qv_arr, src1, dst1, zeros_nd)


def _proj0_body(d, h_ref, w_ref, b_ref, k_ref, qv_ref, s_ref):
    acc = jnp.dot(h_ref[...], w_ref[...],
                  preferred_element_type=jnp.float32) + b_ref[...]
    k_ref[...] = acc[:, :d]
    qv_ref[...] = acc[:, d:3 * d]
    s_ref[...] = acc[:, 3 * d:]


def _proj0(x, w_all, b_all):
    n, d = x.shape
    gsteps = 5
    br = n // gsteps
    return pl.pallas_call(
        functools.partial(_proj0_body, d),
        grid=(gsteps,),
        in_specs=[
            pl.BlockSpec((br, d), lambda i: (i, 0)),
            pl.BlockSpec((d, 4 * d), lambda i: (0, 0)),
            pl.BlockSpec((1, 4 * d), lambda i: (0, 0)),
        ],
        out_specs=[
            pl.BlockSpec((br, d), lambda i: (i, 0)),
            pl.BlockSpec((br, 2 * d), lambda i: (i, 0)),
            pl.BlockSpec((br, d), lambda i: (i, 0)),
        ],
        out_shape=[
            jax.ShapeDtypeStruct((n, d), jnp.float32),
            jax.ShapeDtypeStruct((n, 2 * d), jnp.float32),
            jax.ShapeDtypeStruct((n, d), jnp.float32),
        ],
    )(x, w_all, b_all)


def _proj1_body(d, a0_ref, a1_ref, sp_ref, bp_ref, w_ref, b_ref,
                k_ref, qv_ref, s_ref):
    h = a0_ref[...] + a1_ref[...] + sp_ref[...] + bp_ref[...]
    h = jnp.maximum(h, 0.0)
    acc = jnp.dot(h, w_ref[...], preferred_element_type=jnp.float32) + b_ref[...]
    k_ref[...] = acc[:, :d]
    qv_ref[...] = acc[:, d:3 * d]
    s_ref[...] = acc[:, 3 * d:]


def _proj1(a0, a1, s_prev, b_prev, w_all, b_all):
    n, d = a0.shape
    gsteps = 5
    br = n // gsteps
    return pl.pallas_call(
        functools.partial(_proj1_body, d),
        grid=(gsteps,),
        in_specs=[
            pl.BlockSpec((br, d), lambda i: (i, 0)),
            pl.BlockSpec((br, d), lambda i: (i, 0)),
            pl.BlockSpec((br, d), lambda i: (i, 0)),
            pl.BlockSpec((1, d), lambda i: (0, 0)),
            pl.BlockSpec((d, 4 * d), lambda i: (0, 0)),
            pl.BlockSpec((1, 4 * d), lambda i: (0, 0)),
        ],
        out_specs=[
            pl.BlockSpec((br, d), lambda i: (i, 0)),
            pl.BlockSpec((br, 2 * d), lambda i: (i, 0)),
            pl.BlockSpec((br, d), lambda i: (i, 0)),
        ],
        out_shape=[
            jax.ShapeDtypeStruct((n, d), jnp.float32),
            jax.ShapeDtypeStruct((n, 2 * d), jnp.float32),
            jax.ShapeDtypeStruct((n, d), jnp.float32),
        ],
    )(a0, a1, s_prev, b_prev, w_all, b_all)


def _pool_head_body(g, a0_ref, a1_ref, sp_ref, bp_ref, batch_ref,
                    w1_ref, b1_ref, w2_ref, b2_ref, w3_ref, b3_ref,
                    out_ref, gmax_ref):
    h = a0_ref[...] + a1_ref[...] + sp_ref[...] + bp_ref[...]
    bt = batch_ref[...]
    seg = lax.broadcasted_iota(jnp.int32, (1, g), 1)
    oh = (bt == seg).astype(jnp.float32)
    gsum = lax.dot_general(oh, h, (((0,), (0,)), ((), ())),
                           preferred_element_type=jnp.float32)
    ones_col = jnp.ones((h.shape[0], 1), jnp.float32)
    cnt = lax.dot_general(oh, ones_col, (((0,), (0,)), ((), ())),
                          preferred_element_type=jnp.float32)
    gmean = gsum / jnp.maximum(cnt, 1.0)

    def seg_max(gi, carry):
        mask = bt == gi
        mg = jnp.max(jnp.where(mask, h, -jnp.inf), axis=0, keepdims=True)
        gmax_ref[pl.ds(gi, 1), :] = mg
        return carry

    lax.fori_loop(0, g, seg_max, 0, unroll=False)
    gmax = gmax_ref[...]
    f = jnp.concatenate([gmax, gmean], axis=1)
    o = jnp.maximum(jnp.dot(f, w1_ref[...],
                            preferred_element_type=jnp.float32) + b1_ref[...],
                    0.0)
    o = jnp.maximum(jnp.dot(o, w2_ref[...],
                            preferred_element_type=jnp.float32) + b2_ref[...],
                    0.0)
    out_ref[...] = jnp.dot(o, w3_ref[...],
                           preferred_element_type=jnp.float32) + b3_ref[...]


def _pool_head(a0, a1, s_prev, b_prev, batch2d, g,
               w1, b1, w2, b2, w3p, b3p):
    n, d = a0.shape
    return pl.pallas_call(
        functools.partial(_pool_head_body, g),
        out_shape=jax.ShapeDtypeStruct((g, d), jnp.float32),
        scratch_shapes=[pltpu.VMEM((g, d), jnp.float32)],
    )(a0, a1, s_prev, b_prev, batch2d, w1, b1, w2, b2, w3p, b3p)


def kernel(x, edge_index, batch, params):
    n, d = x.shape
    src1 = edge_index[0].astype(jnp.int32)
    dst1 = edge_index[1].astype(jnp.int32)
    zeros_nd = jnp.zeros((n, d), jnp.float32)

    num_layers = len([k for k in params if k.startswith('conv')])
    h_k = None
    h_qv = None
    h_s = None
    a0 = a1 = None
    b_prev = None
    for li in range(num_layers):
        p = params['conv%d' % li]
        w_all = jnp.concatenate([p['Wk'], p['Wq'], p['Wv'], p['Ws']], axis=1)
        b_all = jnp.concatenate(
            [p['bk'], p['bq'], p['bv'],
             jnp.zeros_like(p['bk'])]).reshape(1, -1)
        if li == 0:
            h_k, h_qv, h_s = _proj0(x, w_all, b_all)
        else:
            h_k, h_qv, h_s = _proj1(a0, a1, h_s, b_prev, w_all, b_all)
        a0, a1 = _edge_stage(h_k, h_qv, src1, dst1, zeros_nd)
        b_prev = p['b'].reshape(1, -1)

    hp = params['head']
    seg_count = 64
    bn_scale = 1.0 / jnp.sqrt(1.0 + 1e-5)
    w1 = hp['W1'] * (hp['g1'] * bn_scale)[None, :]
    b1 = (hp['b1'] * hp['g1'] * bn_scale + hp['be1']).reshape(1, -1)
    w2 = hp['W2'] * (hp['g2'] * bn_scale)[None, :]
    b2 = (hp['b2'] * hp['g2'] * bn_scale + hp['be2']).reshape(1, -1)
    classes = hp['W3'].shape[1]
    w3p = jnp.zeros((hp['W3'].shape[0], d), jnp.float32)
    w3p = w3p.at[:, :classes].set(hp['W3'])
    b3p = jnp.zeros((1, d), jnp.float32)
    b3p = b3p.at[:, :classes].set(hp['b3'][None, :])

    batch2d = batch.astype(jnp.int32).reshape(-1, 1)
    out = _pool_head(a0, a1, h_s, b_prev, batch2d, seg_count,
                     w1, b1, w2, b2, w3p, b3p)
    return out[:, :classes]


# speedup vs baseline: 1.5133x; 1.1194x over previous
import functools

import jax
import jax.numpy as jnp
from jax import lax
from jax.experimental import pallas as pl
from jax.experimental.pallas import tpu as pltpu
from jax.experimental.pallas import tpu_sc as plsc

_NC = 2
_NS = 16
_LANES = 16


def _pick_chunk(per_worker):
    for b in range(40, 7, -8):
        if per_worker % b == 0:
            return b
    return None


def _edge_stage(k_arr, qv_arr, src1, dst1, zeros_nd):
    n, d = k_arr.shape
    e = src1.shape[0]
    nw = _NC * _NS
    per_w = e // nw
    b = _pick_chunk(per_w)
    chunks = per_w // b
    rows_main = (n // _NS) & ~7
    rows_rem = n - _NS * rows_main
    nchunk = d // _LANES

    mesh = plsc.VectorSubcoreMesh(
        core_axis_name="c", subcore_axis_name="s",
        num_cores=_NC, num_subcores=_NS)

    @functools.partial(
        pl.kernel,
        mesh=mesh,
        out_type=(
            jax.ShapeDtypeStruct((n, d), jnp.float32),
            jax.ShapeDtypeStruct((n, d), jnp.float32),
        ),
        scratch_types=[
            pltpu.VMEM((b,), jnp.int32),
            pltpu.VMEM((b,), jnp.int32),
            pltpu.VMEM((b,), jnp.int32),
            pltpu.VMEM((b,), jnp.int32),
            pltpu.VMEM((b,), jnp.int32),
            pltpu.VMEM((b,), jnp.int32),
            pltpu.VMEM((b,), jnp.int32),
            pltpu.VMEM((b,), jnp.int32),
            pltpu.VMEM((b,), jnp.int32),
            pltpu.VMEM((b,), jnp.int32),
            pltpu.VMEM((b, 2 * d), jnp.float32),
            pltpu.VMEM((b, 2 * d), jnp.float32),
            pltpu.VMEM((b, d), jnp.float32),
            pltpu.VMEM((b, d), jnp.float32),
            pltpu.VMEM((b, d), jnp.float32),
            pltpu.VMEM((b, d), jnp.float32),
            pltpu.VMEM_SHARED((n, d), jnp.float32),
            pltpu.SemaphoreType.DMA,
            pltpu.SemaphoreType.DMA,
            pltpu.SemaphoreType.DMA,
            pltpu.SemaphoreType.DMA,
            pltpu.SemaphoreType.DMA,
            pltpu.SemaphoreType.DMA,
            pltpu.SemaphoreType.DMA,
            pltpu.SemaphoreType.DMA,
            pltpu.SemaphoreType.DMA,
        ],
    )
    def edge_kernel(k_hbm, qv_hbm, src_hbm, dst_hbm, zero_hbm,
                    out0, out1,
                    sb0, sb1, sb2, sb3, sb4,
                    db0, db1, db2, db3, db4,
                    qvb0, qvb1, kb0, kb1, ob0, ob1, agg_sh,
                    is0, is1, is2, is3, is4,
                    gsem0, gsem1, ssem0, ssem1):
        c = lax.axis_index("c")
        s = lax.axis_index("s")
        wid = s * _NC + c
        r0 = s * rows_main
        sb = (sb0, sb1, sb2, sb3, sb4)
        db = (db0, db1, db2, db3, db4)
        isem = (is0, is1, is2, is3, is4)
        qvb = (qvb0, qvb1)
        kb = (kb0, kb1)
        ob = (ob0, ob1)
        gsem = (gsem0, gsem1)
        ssem = (ssem0, ssem1)
        pltpu.sync_copy(zero_hbm.at[pl.ds(r0, rows_main)],
                        agg_sh.at[pl.ds(r0, rows_main)])
        if rows_rem:
            @pl.when(s == _NS - 1)
            def _():
                rr = _NS * rows_main
                pltpu.sync_copy(zero_hbm.at[pl.ds(rr, rows_rem)],
                                agg_sh.at[pl.ds(rr, rows_rem)])
        plsc.subcore_barrier()

        base0 = wid * (chunks * b)

        def fetch_idx(j, isl):
            pltpu.async_copy(src_hbm.at[pl.ds(base0 + j * b, b)], sb[isl],
                             isem[isl])
            pltpu.async_copy(dst_hbm.at[pl.ds(base0 + j * b, b)], db[isl],
                             isem[isl])

        def wait_idx(j, isl):
            pltpu.make_async_copy(src_hbm.at[pl.ds(base0 + j * b, b)],
                                  sb[isl], isem[isl]).wait()
            pltpu.make_async_copy(dst_hbm.at[pl.ds(base0 + j * b, b)],
                                  db[isl], isem[isl]).wait()

        def issue_gathers(isl, dsl):
            pltpu.async_copy(qv_hbm.at[sb[isl]], qvb[dsl], gsem[dsl])
            pltpu.async_copy(k_hbm.at[db[isl]], kb[dsl], gsem[dsl])

        def wait_gathers(isl, dsl):
            pltpu.make_async_copy(qv_hbm.at[sb[isl]], qvb[dsl],
                                  gsem[dsl]).wait()
            pltpu.make_async_copy(k_hbm.at[db[isl]], kb[dsl],
                                  gsem[dsl]).wait()

        def drain_scatter(dsl, isl):
            pltpu.make_async_copy(ob[dsl], agg_sh.at[db[isl]],
                                  ssem[dsl]).wait()

        def turn(j, t):
            dsl = t % 2
            @pl.when(j + 1 < chunks)
            def _():
                wait_idx(j + 1, (t + 1) % 5)
                issue_gathers((t + 1) % 5, 1 - dsl)

            @pl.when(j >= 2)
            def _():
                drain_scatter(dsl, (t - 2) % 5)

            @pl.when(j + 3 < chunks)
            def _():
                fetch_idx(j + 3, (t + 3) % 5)

            wait_gathers(t % 5, dsl)

            assert nchunk & (nchunk - 1) == 0
            csh = nchunk.bit_length() - 1

            @plsc.parallel_loop(0, b * nchunk, 1, unroll=4)
            def row_body(i):
                r = i >> csh
                c16 = (i & (nchunk - 1)) * _LANES
                slc = pl.ds(c16, _LANES)
                tt = kb[dsl][r, slc] + qvb[dsl][r, slc]
                v = qvb[dsl][r, pl.ds(d + c16, _LANES)]
                ob[dsl][r, slc] = v / (1.0 + jnp.exp(-tt))

            pltpu.async_copy(ob[dsl], agg_sh.at[db[t % 5]], ssem[dsl],
                             add=True)

        fetch_idx(jnp.int32(0), 0)
        fetch_idx(jnp.int32(1), 1)
        fetch_idx(jnp.int32(2), 2)
        wait_idx(jnp.int32(0), 0)
        issue_gathers(0, 0)

        assert chunks % 10 == 0, chunks

        def dec_body(p, carry):
            for t in range(10):
                turn(10 * p + t, t)
            return carry

        lax.fori_loop(0, chunks // 10, dec_body, 0, unroll=False)
        drain_scatter((chunks - 1) % 2, (chunks - 1) % 5)
        drain_scatter((chunks - 2) % 2, (chunks - 2) % 5)
        plsc.subcore_barrier()

        @pl.when(c == 0)
        def _():
            pltpu.sync_copy(agg_sh.at[pl.ds(r0, rows_main)],
                            out0.at[pl.ds(r0, rows_main)])
            if rows_rem:
                @pl.when(s == _NS - 1)
                def _():
                    rr = _NS * rows_main
                    pltpu.sync_copy(agg_sh.at[pl.ds(rr, rows_rem)],
                                    out0.at[pl.ds(rr, rows_rem)])

        @pl.when(c == 1)
        def _():
            pltpu.sync_copy(agg_sh.at[pl.ds(r0, rows_main)],
                            out1.at[pl.ds(r0, rows_main)])
            if rows_rem:
                @pl.when(s == _NS - 1)
                def _():
                    rr = _NS * rows_main
                    pltpu.sync_copy(agg_sh.at[pl.ds(rr, rows_rem)],
                                    out1.at[pl.ds(rr, rows_rem)])

    return edge_kernel(k_arr, qv_arr, src1, dst1, zeros_nd)


def _proj0_body(d, h_ref, w_ref, b_ref, k_ref, qv_ref, s_ref):
    acc = jnp.dot(h_ref[...], w_ref[...],
                  preferred_element_type=jnp.float32) + b_ref[...]
    k_ref[...] = acc[:, :d]
    qv_ref[...] = acc[:, d:3 * d]
    s_ref[...] = acc[:, 3 * d:]


def _proj0(x, w_all, b_all):
    n, d = x.shape
    gsteps = 5
    br = n // gsteps
    return pl.pallas_call(
        functools.partial(_proj0_body, d),
        grid=(gsteps,),
        in_specs=[
            pl.BlockSpec((br, d), lambda i: (i, 0)),
            pl.BlockSpec((d, 4 * d), lambda i: (0, 0)),
            pl.BlockSpec((1, 4 * d), lambda i: (0, 0)),
        ],
        out_specs=[
            pl.BlockSpec((br, d), lambda i: (i, 0)),
            pl.BlockSpec((br, 2 * d), lambda i: (i, 0)),
            pl.BlockSpec((br, d), lambda i: (i, 0)),
        ],
        out_shape=[
            jax.ShapeDtypeStruct((n, d), jnp.float32),
            jax.ShapeDtypeStruct((n, 2 * d), jnp.float32),
            jax.ShapeDtypeStruct((n, d), jnp.float32),
        ],
    )(x, w_all, b_all)


def _proj1_body(d, a0_ref, a1_ref, sp_ref, bp_ref, w_ref, b_ref,
                k_ref, qv_ref, s_ref):
    h = a0_ref[...] + a1_ref[...] + sp_ref[...] + bp_ref[...]
    h = jnp.maximum(h, 0.0)
    acc = jnp.dot(h, w_ref[...], preferred_element_type=jnp.float32) + b_ref[...]
    k_ref[...] = acc[:, :d]
    qv_ref[...] = acc[:, d:3 * d]
    s_ref[...] = acc[:, 3 * d:]


def _proj1(a0, a1, s_prev, b_prev, w_all, b_all):
    n, d = a0.shape
    gsteps = 5
    br = n // gsteps
    return pl.pallas_call(
        functools.partial(_proj1_body, d),
        grid=(gsteps,),
        in_specs=[
            pl.BlockSpec((br, d), lambda i: (i, 0)),
            pl.BlockSpec((br, d), lambda i: (i, 0)),
            pl.BlockSpec((br, d), lambda i: (i, 0)),
            pl.BlockSpec((1, d), lambda i: (0, 0)),
            pl.BlockSpec((d, 4 * d), lambda i: (0, 0)),
            pl.BlockSpec((1, 4 * d), lambda i: (0, 0)),
        ],
        out_specs=[
            pl.BlockSpec((br, d), lambda i: (i, 0)),
            pl.BlockSpec((br, 2 * d), lambda i: (i, 0)),
            pl.BlockSpec((br, d), lambda i: (i, 0)),
        ],
        out_shape=[
            jax.ShapeDtypeStruct((n, d), jnp.float32),
            jax.ShapeDtypeStruct((n, 2 * d), jnp.float32),
            jax.ShapeDtypeStruct((n, d), jnp.float32),
        ],
    )(a0, a1, s_prev, b_prev, w_all, b_all)


def _pool_head_body(g, a0_ref, a1_ref, sp_ref, bp_ref, batch_ref,
                    w1_ref, b1_ref, w2_ref, b2_ref, w3_ref, b3_ref,
                    out_ref, gmax_ref):
    h = a0_ref[...] + a1_ref[...] + sp_ref[...] + bp_ref[...]
    bt = batch_ref[...]
    seg = lax.broadcasted_iota(jnp.int32, (1, g), 1)
    oh = (bt == seg).astype(jnp.float32)
    gsum = lax.dot_general(oh, h, (((0,), (0,)), ((), ())),
                           preferred_element_type=jnp.float32)
    ones_col = jnp.ones((h.shape[0], 1), jnp.float32)
    cnt = lax.dot_general(oh, ones_col, (((0,), (0,)), ((), ())),
                          preferred_element_type=jnp.float32)
    gmean = gsum / jnp.maximum(cnt, 1.0)

    def seg_max(gi, carry):
        mask = bt == gi
        mg = jnp.max(jnp.where(mask, h, -jnp.inf), axis=0, keepdims=True)
        gmax_ref[pl.ds(gi, 1), :] = mg
        return carry

    lax.fori_loop(0, g, seg_max, 0, unroll=False)
    gmax = gmax_ref[...]
    f = jnp.concatenate([gmax, gmean], axis=1)
    o = jnp.maximum(jnp.dot(f, w1_ref[...],
                            preferred_element_type=jnp.float32) + b1_ref[...],
                    0.0)
    o = jnp.maximum(jnp.dot(o, w2_ref[...],
                            preferred_element_type=jnp.float32) + b2_ref[...],
                    0.0)
    out_ref[...] = jnp.dot(o, w3_ref[...],
                           preferred_element_type=jnp.float32) + b3_ref[...]


def _pool_head(a0, a1, s_prev, b_prev, batch2d, g,
               w1, b1, w2, b2, w3p, b3p):
    n, d = a0.shape
    return pl.pallas_call(
        functools.partial(_pool_head_body, g),
        out_shape=jax.ShapeDtypeStruct((g, d), jnp.float32),
        scratch_shapes=[pltpu.VMEM((g, d), jnp.float32)],
    )(a0, a1, s_prev, b_prev, batch2d, w1, b1, w2, b2, w3p, b3p)


def kernel(x, edge_index, batch, params):
    n, d = x.shape
    src1 = edge_index[0].astype(jnp.int32)
    dst1 = edge_index[1].astype(jnp.int32)
    zeros_nd = jnp.zeros((n, d), jnp.float32)

    num_layers = len([k for k in params if k.startswith('conv')])
    h_k = None
    h_qv = None
    h_s = None
    a0 = a1 = None
    b_prev = None
    for li in range(num_layers):
        p = params['conv%d' % li]
        w_all = jnp.concatenate([p['Wk'], p['Wq'], p['Wv'], p['Ws']], axis=1)
        b_all = jnp.concatenate(
            [p['bk'], p['bq'], p['bv'],
             jnp.zeros_like(p['bk'])]).reshape(1, -1)
        if li == 0:
            h_k, h_qv, h_s = _proj0(x, w_all, b_all)
        else:
            h_k, h_qv, h_s = _proj1(a0, a1, h_s, b_prev, w_all, b_all)
        a0, a1 = _edge_stage(h_k, h_qv, src1, dst1, zeros_nd)
        b_prev = p['b'].reshape(1, -1)

    hp = params['head']
    seg_count = 64
    bn_scale = 1.0 / jnp.sqrt(1.0 + 1e-5)
    w1 = hp['W1'] * (hp['g1'] * bn_scale)[None, :]
    b1 = (hp['b1'] * hp['g1'] * bn_scale + hp['be1']).reshape(1, -1)
    w2 = hp['W2'] * (hp['g2'] * bn_scale)[None, :]
    b2 = (hp['b2'] * hp['g2'] * bn_scale + hp['be2']).reshape(1, -1)
    classes = hp['W3'].shape[1]
    w3p = jnp.zeros((hp['W3'].shape[0], d), jnp.float32)
    w3p = w3p.at[:, :classes].set(hp['W3'])
    b3p = jnp.zeros((1, d), jnp.float32)
    b3p = b3p.at[:, :classes].set(hp['b3'][None, :])

    batch2d = batch.astype(jnp.int32).reshape(-1, 1)
    out = _pool_head(a0, a1, h_s, b_prev, batch2d, seg_count,
                     w1, b1, w2, b2, w3p, b3p)
    return out[:, :classes]


# speedup vs baseline: 1.5651x; 1.0342x over previous
import functools

import jax
import jax.numpy as jnp
from jax import lax
from jax.experimental import pallas as pl
from jax.experimental.pallas import tpu as pltpu
from jax.experimental.pallas import tpu_sc as plsc

_NC = 2
_NS = 16
_LANES = 16


def _pick_chunk(per_worker):
    for b in range(40, 7, -8):
        if per_worker % b == 0:
            return b
    return None


def _edge_stage(k_arr, qv_arr, src1, dst1, zeros_nd):
    n, d = k_arr.shape
    e = src1.shape[0]
    nw = _NC * _NS
    per_w = e // nw
    b = _pick_chunk(per_w)
    chunks = per_w // b
    rows_main = (n // _NS) & ~7
    rows_rem = n - _NS * rows_main
    nchunk = d // _LANES

    mesh = plsc.VectorSubcoreMesh(
        core_axis_name="c", subcore_axis_name="s",
        num_cores=_NC, num_subcores=_NS)

    @functools.partial(
        pl.kernel,
        mesh=mesh,
        out_type=(
            jax.ShapeDtypeStruct((n, d), jnp.float32),
            jax.ShapeDtypeStruct((n, d), jnp.float32),
        ),
        scratch_types=[
            pltpu.VMEM((b,), jnp.int32),
            pltpu.VMEM((b,), jnp.int32),
            pltpu.VMEM((b,), jnp.int32),
            pltpu.VMEM((b,), jnp.int32),
            pltpu.VMEM((b,), jnp.int32),
            pltpu.VMEM((b,), jnp.int32),
            pltpu.VMEM((b,), jnp.int32),
            pltpu.VMEM((b,), jnp.int32),
            pltpu.VMEM((b,), jnp.int32),
            pltpu.VMEM((b,), jnp.int32),
            pltpu.VMEM((b, 2 * d), jnp.float32),
            pltpu.VMEM((b, 2 * d), jnp.float32),
            pltpu.VMEM((b, d), jnp.float32),
            pltpu.VMEM((b, d), jnp.float32),
            pltpu.VMEM((b, d), jnp.float32),
            pltpu.VMEM((b, d), jnp.float32),
            pltpu.VMEM_SHARED((n, d), jnp.float32),
            pltpu.SemaphoreType.DMA,
            pltpu.SemaphoreType.DMA,
            pltpu.SemaphoreType.DMA,
            pltpu.SemaphoreType.DMA,
            pltpu.SemaphoreType.DMA,
            pltpu.SemaphoreType.DMA,
            pltpu.SemaphoreType.DMA,
            pltpu.SemaphoreType.DMA,
            pltpu.SemaphoreType.DMA,
        ],
    )
    def edge_kernel(k_hbm, qv_hbm, src_hbm, dst_hbm, zero_hbm,
                    out0, out1,
                    sb0, sb1, sb2, sb3, sb4,
                    db0, db1, db2, db3, db4,
                    qvb0, qvb1, kb0, kb1, ob0, ob1, agg_sh,
                    is0, is1, is2, is3, is4,
                    gsem0, gsem1, ssem0, ssem1):
        c = lax.axis_index("c")
        s = lax.axis_index("s")
        wid = s * _NC + c
        r0 = s * rows_main
        sb = (sb0, sb1, sb2, sb3, sb4)
        db = (db0, db1, db2, db3, db4)
        isem = (is0, is1, is2, is3, is4)
        qvb = (qvb0, qvb1)
        kb = (kb0, kb1)
        ob = (ob0, ob1)
        gsem = (gsem0, gsem1)
        ssem = (ssem0, ssem1)
        pltpu.sync_copy(zero_hbm.at[pl.ds(r0, rows_main)],
                        agg_sh.at[pl.ds(r0, rows_main)])
        if rows_rem:
            @pl.when(s == _NS - 1)
            def _():
                rr = _NS * rows_main
                pltpu.sync_copy(zero_hbm.at[pl.ds(rr, rows_rem)],
                                agg_sh.at[pl.ds(rr, rows_rem)])
        plsc.subcore_barrier()

        base0 = wid * (chunks * b)

        def fetch_idx(j, isl):
            pltpu.async_copy(src_hbm.at[pl.ds(base0 + j * b, b)], sb[isl],
                             isem[isl])
            pltpu.async_copy(dst_hbm.at[pl.ds(base0 + j * b, b)], db[isl],
                             isem[isl])

        def wait_idx(j, isl):
            pltpu.make_async_copy(src_hbm.at[pl.ds(base0 + j * b, b)],
                                  sb[isl], isem[isl]).wait()
            pltpu.make_async_copy(dst_hbm.at[pl.ds(base0 + j * b, b)],
                                  db[isl], isem[isl]).wait()

        def issue_gathers(isl, dsl):
            pltpu.async_copy(qv_hbm.at[sb[isl]], qvb[dsl], gsem[dsl])
            pltpu.async_copy(k_hbm.at[db[isl]], kb[dsl], gsem[dsl])

        def wait_gathers(isl, dsl):
            pltpu.make_async_copy(qv_hbm.at[sb[isl]], qvb[dsl],
                                  gsem[dsl]).wait()
            pltpu.make_async_copy(k_hbm.at[db[isl]], kb[dsl],
                                  gsem[dsl]).wait()

        def drain_scatter(dsl, isl):
            pltpu.make_async_copy(ob[dsl], agg_sh.at[db[isl]],
                                  ssem[dsl]).wait()

        def turn(j, t):
            dsl = t % 2
            @pl.when(j + 1 < chunks)
            def _():
                wait_idx(j + 1, (t + 1) % 5)
                issue_gathers((t + 1) % 5, 1 - dsl)

            @pl.when(j >= 2)
            def _():
                drain_scatter(dsl, (t - 2) % 5)

            @pl.when(j + 3 < chunks)
            def _():
                fetch_idx(j + 3, (t + 3) % 5)

            wait_gathers(t % 5, dsl)

            assert nchunk & (nchunk - 1) == 0
            csh = nchunk.bit_length() - 1

            @plsc.parallel_loop(0, b * nchunk, 1, unroll=8)
            def row_body(i):
                r = i >> csh
                c16 = (i & (nchunk - 1)) * _LANES
                slc = pl.ds(c16, _LANES)
                tt = kb[dsl][r, slc] + qvb[dsl][r, slc]
                v = qvb[dsl][r, pl.ds(d + c16, _LANES)]
                ob[dsl][r, slc] = v / (1.0 + jnp.exp(-tt))

            pltpu.async_copy(ob[dsl], agg_sh.at[db[t % 5]], ssem[dsl],
                             add=True)

        fetch_idx(jnp.int32(0), 0)
        fetch_idx(jnp.int32(1), 1)
        fetch_idx(jnp.int32(2), 2)
        wait_idx(jnp.int32(0), 0)
        issue_gathers(0, 0)

        assert chunks % 10 == 0, chunks

        def dec_body(p, carry):
            for t in range(10):
                turn(10 * p + t, t)
            return carry

        lax.fori_loop(0, chunks // 10, dec_body, 0, unroll=False)
        drain_scatter((chunks - 1) % 2, (chunks - 1) % 5)
        drain_scatter((chunks - 2) % 2, (chunks - 2) % 5)
        plsc.subcore_barrier()

        @pl.when(c == 0)
        def _():
            pltpu.sync_copy(agg_sh.at[pl.ds(r0, rows_main)],
                            out0.at[pl.ds(r0, rows_main)])
            if rows_rem:
                @pl.when(s == _NS - 1)
                def _():
                    rr = _NS * rows_main
                    pltpu.sync_copy(agg_sh.at[pl.ds(rr, rows_rem)],
                                    out0.at[pl.ds(rr, rows_rem)])

        @pl.when(c == 1)
        def _():
            pltpu.sync_copy(agg_sh.at[pl.ds(r0, rows_main)],
                            out1.at[pl.ds(r0, rows_main)])
            if rows_rem:
                @pl.when(s == _NS - 1)
                def _():
                    rr = _NS * rows_main
                    pltpu.sync_copy(agg_sh.at[pl.ds(rr, rows_rem)],
                                    out1.at[pl.ds(rr, rows_rem)])

    return edge_kernel(k_arr, qv_arr, src1, dst1, zeros_nd)


def _proj0_body(d, h_ref, w_ref, b_ref, k_ref, qv_ref, s_ref):
    acc = jnp.dot(h_ref[...], w_ref[...],
                  preferred_element_type=jnp.float32) + b_ref[...]
    k_ref[...] = acc[:, :d]
    qv_ref[...] = acc[:, d:3 * d]
    s_ref[...] = acc[:, 3 * d:]


def _proj0(x, w_all, b_all):
    n, d = x.shape
    gsteps = 5
    br = n // gsteps
    return pl.pallas_call(
        functools.partial(_proj0_body, d),
        grid=(gsteps,),
        in_specs=[
            pl.BlockSpec((br, d), lambda i: (i, 0)),
            pl.BlockSpec((d, 4 * d), lambda i: (0, 0)),
            pl.BlockSpec((1, 4 * d), lambda i: (0, 0)),
        ],
        out_specs=[
            pl.BlockSpec((br, d), lambda i: (i, 0)),
            pl.BlockSpec((br, 2 * d), lambda i: (i, 0)),
            pl.BlockSpec((br, d), lambda i: (i, 0)),
        ],
        out_shape=[
            jax.ShapeDtypeStruct((n, d), jnp.float32),
            jax.ShapeDtypeStruct((n, 2 * d), jnp.float32),
            jax.ShapeDtypeStruct((n, d), jnp.float32),
        ],
    )(x, w_all, b_all)


def _proj1_body(d, a0_ref, a1_ref, sp_ref, bp_ref, w_ref, b_ref,
                k_ref, qv_ref, s_ref):
    h = a0_ref[...] + a1_ref[...] + sp_ref[...] + bp_ref[...]
    h = jnp.maximum(h, 0.0)
    acc = jnp.dot(h, w_ref[...], preferred_element_type=jnp.float32) + b_ref[...]
    k_ref[...] = acc[:, :d]
    qv_ref[...] = acc[:, d:3 * d]
    s_ref[...] = acc[:, 3 * d:]


def _proj1(a0, a1, s_prev, b_prev, w_all, b_all):
    n, d = a0.shape
    gsteps = 5
    br = n // gsteps
    return pl.pallas_call(
        functools.partial(_proj1_body, d),
        grid=(gsteps,),
        in_specs=[
            pl.BlockSpec((br, d), lambda i: (i, 0)),
            pl.BlockSpec((br, d), lambda i: (i, 0)),
            pl.BlockSpec((br, d), lambda i: (i, 0)),
            pl.BlockSpec((1, d), lambda i: (0, 0)),
            pl.BlockSpec((d, 4 * d), lambda i: (0, 0)),
            pl.BlockSpec((1, 4 * d), lambda i: (0, 0)),
        ],
        out_specs=[
            pl.BlockSpec((br, d), lambda i: (i, 0)),
            pl.BlockSpec((br, 2 * d), lambda i: (i, 0)),
            pl.BlockSpec((br, d), lambda i: (i, 0)),
        ],
        out_shape=[
            jax.ShapeDtypeStruct((n, d), jnp.float32),
            jax.ShapeDtypeStruct((n, 2 * d), jnp.float32),
            jax.ShapeDtypeStruct((n, d), jnp.float32),
        ],
    )(a0, a1, s_prev, b_prev, w_all, b_all)


def _pool_head_body(g, a0_ref, a1_ref, sp_ref, bp_ref, batch_ref,
                    w1_ref, b1_ref, w2_ref, b2_ref, w3_ref, b3_ref,
                    out_ref, gmax_ref):
    h = a0_ref[...] + a1_ref[...] + sp_ref[...] + bp_ref[...]
    bt = batch_ref[...]
    seg = lax.broadcasted_iota(jnp.int32, (1, g), 1)
    oh = (bt == seg).astype(jnp.float32)
    gsum = lax.dot_general(oh, h, (((0,), (0,)), ((), ())),
                           preferred_element_type=jnp.float32)
    ones_col = jnp.ones((h.shape[0], 1), jnp.float32)
    cnt = lax.dot_general(oh, ones_col, (((0,), (0,)), ((), ())),
                          preferred_element_type=jnp.float32)
    gmean = gsum / jnp.maximum(cnt, 1.0)

    def seg_max(gi, carry):
        mask = bt == gi
        mg = jnp.max(jnp.where(mask, h, -jnp.inf), axis=0, keepdims=True)
        gmax_ref[pl.ds(gi, 1), :] = mg
        return carry

    lax.fori_loop(0, g, seg_max, 0, unroll=False)
    gmax = gmax_ref[...]
    f = jnp.concatenate([gmax, gmean], axis=1)
    o = jnp.maximum(jnp.dot(f, w1_ref[...],
                            preferred_element_type=jnp.float32) + b1_ref[...],
                    0.0)
    o = jnp.maximum(jnp.dot(o, w2_ref[...],
                            preferred_element_type=jnp.float32) + b2_ref[...],
                    0.0)
    out_ref[...] = jnp.dot(o, w3_ref[...],
                           preferred_element_type=jnp.float32) + b3_ref[...]


def _pool_head(a0, a1, s_prev, b_prev, batch2d, g,
               w1, b1, w2, b2, w3p, b3p):
    n, d = a0.shape
    return pl.pallas_call(
        functools.partial(_pool_head_body, g),
        out_shape=jax.ShapeDtypeStruct((g, d), jnp.float32),
        scratch_shapes=[pltpu.VMEM((g, d), jnp.float32)],
    )(a0, a1, s_prev, b_prev, batch2d, w1, b1, w2, b2, w3p, b3p)


def kernel(x, edge_index, batch, params):
    n, d = x.shape
    src1 = edge_index[0].astype(jnp.int32)
    dst1 = edge_index[1].astype(jnp.int32)
    zeros_nd = jnp.zeros((n, d), jnp.float32)

    num_layers = len([k for k in params if k.startswith('conv')])
    h_k = None
    h_qv = None
    h_s = None
    a0 = a1 = None
    b_prev = None
    for li in range(num_layers):
        p = params['conv%d' % li]
        w_all = jnp.concatenate([p['Wk'], p['Wq'], p['Wv'], p['Ws']], axis=1)
        b_all = jnp.concatenate(
            [p['bk'], p['bq'], p['bv'],
             jnp.zeros_like(p['bk'])]).reshape(1, -1)
        if li == 0:
            h_k, h_qv, h_s = _proj0(x, w_all, b_all)
        else:
            h_k, h_qv, h_s = _proj1(a0, a1, h_s, b_prev, w_all, b_all)
        a0, a1 = _edge_stage(h_k, h_qv, src1, dst1, zeros_nd)
        b_prev = p['b'].reshape(1, -1)

    hp = params['head']
    seg_count = 64
    bn_scale = 1.0 / jnp.sqrt(1.0 + 1e-5)
    w1 = hp['W1'] * (hp['g1'] * bn_scale)[None, :]
    b1 = (hp['b1'] * hp['g1'] * bn_scale + hp['be1']).reshape(1, -1)
    w2 = hp['W2'] * (hp['g2'] * bn_scale)[None, :]
    b2 = (hp['b2'] * hp['g2'] * bn_scale + hp['be2']).reshape(1, -1)
    classes = hp['W3'].shape[1]
    w3p = jnp.zeros((hp['W3'].shape[0], d), jnp.float32)
    w3p = w3p.at[:, :classes].set(hp['W3'])
    b3p = jnp.zeros((1, d), jnp.float32)
    b3p = b3p.at[:, :classes].set(hp['b3'][None, :])

    batch2d = batch.astype(jnp.int32).reshape(-1, 1)
    out = _pool_head(a0, a1, h_s, b_prev, batch2d, seg_count,
                     w1, b1, w2, b2, w3p, b3p)
    return out[:, :classes]
